# Initial kernel scaffold; baseline (speedup 1.0000x reference)
import jax
import jax.numpy as jnp
from jax.experimental import pallas as pl


def kernel(node_features, edge_features, edge_index, params):
    raise NotImplementedError("write your pallas kernel here")


# speedup vs baseline: 3.1937x; 3.1937x over previous
import functools

import jax
import jax.numpy as jnp
from jax import lax
from jax.experimental import pallas as pl
from jax.experimental.pallas import tpu as pltpu
from jax.experimental.pallas import tpu_sc as plsc

N = 50000
E = 800000
H = 64
NPAD = 50048
EPAD = 819200
ER = EPAD // 128
BN = 2000
BE = 8000
HALF = 25000
TBL = 25088
PREC = lax.Precision.HIGHEST

_f32 = jnp.float32
_i32 = jnp.int32


def _lr(x):
    return jnp.where(x >= 0, x, 0.2 * x)


def _rows(shape):
    nd = len(shape)
    return pl.BlockSpec(shape, lambda i: (i,) + (0,) * (nd - 1))


def _full(shape):
    nd = len(shape)
    return pl.BlockSpec(shape, lambda i: (0,) * nd)


def _dot(a, b):
    return jnp.dot(a, b, precision=PREC)


def _enc_body(x, w1, b1, w2, b2, o):
    o[...] = _dot(jax.nn.relu(_dot(x[...], w1[...]) + b1[...]), w2[...]) + b2[...]


def _enc_edges_body(x, w1, b1, w2, b2, we, ae, oe, ob):
    e = _dot(jax.nn.relu(_dot(x[...], w1[...]) + b1[...]), w2[...]) + b2[...]
    oe[...] = e
    ob[...] = jnp.sum(_lr(_dot(e, we[...])) * ae[...], -1, keepdims=True)


def _node_pre_body(h, wq, wk, wm, aqr, akr, oaq, oak, om):
    hh = h[...]
    oaq[...] = jnp.sum(_lr(_dot(hh, wq[...])) * aqr[...], -1, keepdims=True)
    oak[...] = jnp.sum(_lr(_dot(hh, wk[...])) * akr[...], -1, keepdims=True)
    om[...] = _dot(hh, wm[...])


def _node_post_body(h, agg, sr, aqc, wself, bself, g, b, wa, wb, ohn, oa, ob):
    hh = h[...]
    a = agg[...] / (sr[...] + 1e-12 * jnp.exp(-aqc[...]))
    hn = jax.nn.relu(_dot(hh, wself[...]) + bself[...] + a)
    x = hn + hh
    mu = jnp.mean(x, -1, keepdims=True)
    xc = x - mu
    var = jnp.mean(xc * xc, -1, keepdims=True)
    hn2 = xc / jnp.sqrt(var + 1e-5) * g[...] + b[...]
    ohn[...] = hn2
    oa[...] = _dot(hn2, wa[...])
    ob[...] = _dot(hn2, wb[...])


def _edge_post_beta_body(e, ga, gb, wee, beu, wen, aen, oe, ob):
    en = jax.nn.relu(ga[...] + gb[...] + _dot(e[...], wee[...]) + beu[...]) + e[...]
    oe[...] = en
    ob[...] = jnp.sum(_lr(_dot(en, wen[...])) * aen[...], -1, keepdims=True)


def _edge_post_body(e, ga, gb, wee, beu, oe):
    oe[...] = jax.nn.relu(ga[...] + gb[...] + _dot(e[...], wee[...]) + beu[...]) + e[...]


def _node_final_body(h, ws, wd, ou, ov):
    hh = h[...]
    ou[...] = _dot(hh, ws[...])
    ov[...] = _dot(hh, wd[...])


def _final_body(e, gu, gv, w1e, b1, w2r, b2, os):
    hid = jax.nn.relu(gu[...] + gv[...] + _dot(e[...], w1e[...]) + b1[...])
    os[...] = jax.nn.sigmoid(jnp.sum(hid * w2r[...], -1, keepdims=True) + b2[...])


def _tc(body, grid, in_specs, out_specs, out_shape):
    return pl.pallas_call(
        body, grid=(grid,), in_specs=in_specs, out_specs=out_specs,
        out_shape=out_shape)


_MESH = plsc.VectorSubcoreMesh(core_axis_name="c", subcore_axis_name="s")
_SC_PARAMS = pltpu.CompilerParams(
    needs_layout_passes=False, use_tc_tiling_on_sc=False)


def _sc_pair_gather_body(ta, tb, srcr, dstr, ga, gb,
                         sidx, didx, ra, rb, sem):
    c = lax.axis_index("c")
    s = lax.axis_index("s")
    wid = s * 2 + c

    def chunk(k, carry):
        row0 = wid * 200 + k * 4
        pltpu.sync_copy(srcr.at[pl.ds(row0, 4)], sidx)
        pltpu.sync_copy(dstr.at[pl.ds(row0, 4)], didx)
        hs = []
        for j in range(4):
            hs.append(pltpu.async_copy(ta.at[sidx.at[j]],
                                       ra.at[pl.ds(j * 128, 128)], sem))
            hs.append(pltpu.async_copy(tb.at[didx.at[j]],
                                       rb.at[pl.ds(j * 128, 128)], sem))
        for hnd in hs:
            hnd.wait()
        eb = row0 * 128
        pltpu.sync_copy(ra, ga.at[pl.ds(eb, 512)])
        pltpu.sync_copy(rb, gb.at[pl.ds(eb, 512)])
        return carry

    lax.fori_loop(0, 50, chunk, 0)


_sc_pair_gather = pl.kernel(
    _sc_pair_gather_body,
    out_type=[jax.ShapeDtypeStruct((EPAD, H), _f32),
              jax.ShapeDtypeStruct((EPAD, H), _f32)],
    mesh=_MESH,
    compiler_params=_SC_PARAMS,
    scratch_types=[
        pltpu.VMEM((4, 128), _i32),
        pltpu.VMEM((4, 128), _i32),
        pltpu.VMEM((512, H), _f32),
        pltpu.VMEM((512, H), _f32),
        pltpu.SemaphoreType.DMA,
    ],
)


def _make_sc_attn_body():
    def body(srcr, dstr, betar, akt, hmsg, s_out, agg_out,
             sidx, didx, bv, akv, sv, sflat, lidx, rows, zs,
             s_sh, agg_sh, sem):
        c = lax.axis_index("c")
        s = lax.axis_index("s")
        base = c * HALF
        zero16 = jnp.zeros((16,), _f32)

        def zrow(i, carry):
            for jj in range(4):
                rows[i, pl.ds(jj * 16, 16)] = zero16
            return carry

        lax.fori_loop(0, 256, zrow, 0)

        def zz(i, carry):
            zs[pl.ds(i * 16, 16)] = zero16
            return carry

        lax.fori_loop(0, 98, zz, 0)

        r0 = pl.multiple_of(s * 1568, 8)
        for t in range(6):
            pltpu.sync_copy(rows, agg_sh.at[pl.ds(r0 + t * 256, 256)])
        pltpu.sync_copy(rows.at[pl.ds(0, 32)], agg_sh.at[pl.ds(r0 + 1536, 32)])
        pltpu.sync_copy(zs, s_sh.at[pl.ds(r0, 1568)])
        plsc.subcore_barrier()

        iota16 = lax.iota(_i32, 16)

        def chunk(k, carry):
            row0 = s * 400 + k * 2
            pltpu.sync_copy(srcr.at[pl.ds(row0, 2)], sidx)
            pltpu.sync_copy(dstr.at[pl.ds(row0, 2)], didx)
            pltpu.sync_copy(betar.at[pl.ds(row0, 2)], bv)
            hs = []
            for j in range(2):
                hs.append(pltpu.async_copy(akt.at[sidx.at[j]], akv.at[j], sem))
                hs.append(pltpu.async_copy(hmsg.at[sidx.at[j]],
                                           rows.at[pl.ds(j * 128, 128)], sem))
            for hnd in hs:
                hnd.wait()

            for j in range(2):
                def grp(i, carry2, j=j):
                    dsl = pl.ds(i * 16, 16)
                    lv = akv[j, dsl] + bv[j, dsl]
                    sval = jnp.exp(lv)
                    sv[j, dsl] = sval
                    sflat[pl.ds(j * 128 + i * 16, 16)] = sval
                    d = didx[j, dsl] - base
                    owned = (d >= 0) & (d < HALF)
                    tr = HALF + iota16 + (i % 4) * 16
                    lidx[j, dsl] = jnp.where(owned, d, tr)
                    return carry2

                lax.fori_loop(0, 8, grp, 0)

            def scale(ei, carry2):
                spl = plsc.load_gather(sflat, [jnp.full((16,), ei, _i32)])
                for jj in range(4):
                    dsl = pl.ds(jj * 16, 16)
                    rows[ei, dsl] = rows[ei, dsl] * spl
                return carry2

            lax.fori_loop(0, 256, scale, 0)

            for j in range(2):
                pltpu.sync_copy(rows.at[pl.ds(j * 128, 128)],
                                agg_sh.at[lidx.at[j]], add=True)
                pltpu.sync_copy(sv.at[j], s_sh.at[lidx.at[j]], add=True)
            return carry

        lax.fori_loop(0, 200, chunk, 0)
        plsc.subcore_barrier()

        o0 = pl.multiple_of(s * 1560, 8)
        pltpu.sync_copy(agg_sh.at[pl.ds(o0, 1560)],
                        agg_out.at[pl.ds(base + o0, 1560)])
        pltpu.sync_copy(s_sh.at[pl.ds(o0, 1560)],
                        s_out.at[pl.ds(base + o0, 1560)])

        @pl.when(s == 0)
        def _tail():
            pltpu.sync_copy(agg_sh.at[pl.ds(24960, 40)],
                            agg_out.at[pl.ds(base + 24960, 40)])
            pltpu.sync_copy(s_sh.at[pl.ds(24960, 40)],
                            s_out.at[pl.ds(base + 24960, 40)])

    return body


_sc_attn_agg = pl.kernel(
    _make_sc_attn_body(),
    out_type=[jax.ShapeDtypeStruct((N,), _f32),
              jax.ShapeDtypeStruct((N, H), _f32)],
    mesh=_MESH,
    compiler_params=_SC_PARAMS,
    scratch_types=[
        pltpu.VMEM((2, 128), _i32),
        pltpu.VMEM((2, 128), _i32),
        pltpu.VMEM((2, 128), _f32),
        pltpu.VMEM((2, 128), _f32),
        pltpu.VMEM((2, 128), _f32),
        pltpu.VMEM((256,), _f32),
        pltpu.VMEM((2, 128), _i32),
        pltpu.VMEM((256, H), _f32),
        pltpu.VMEM((1568,), _f32),
        pltpu.VMEM_SHARED((TBL,), _f32),
        pltpu.VMEM_SHARED((TBL, H), _f32),
        pltpu.SemaphoreType.DMA,
    ],
)


def kernel(node_features, edge_features, edge_index, params):
    p = params
    src = edge_index[0].astype(_i32)
    dst = edge_index[1].astype(_i32)
    srcr = jnp.pad(src, (0, EPAD - E)).reshape(ER, 128)
    dstr = jnp.pad(dst, (0, EPAD - E), constant_values=N).reshape(ER, 128)

    h = _tc(_enc_body, N // BN,
            [_rows((BN, 8)), _full((8, H)), _full((1, H)), _full((H, H)),
             _full((1, H))],
            _rows((BN, H)), jax.ShapeDtypeStruct((N, H), _f32))(
        node_features, p['ne_w1'], p['ne_b1'].reshape(1, H),
        p['ne_w2'], p['ne_b2'].reshape(1, H))

    lp0 = p['layers'][0]
    e, beta = _tc(_enc_edges_body, E // BE,
                  [_rows((BE, 8)), _full((8, H)), _full((1, H)), _full((H, H)),
                   _full((1, H)), _full((H, H)), _full((1, H))],
                  [_rows((BE, H)), _rows((BE, 1))],
                  [jax.ShapeDtypeStruct((E, H), _f32),
                   jax.ShapeDtypeStruct((E, 1), _f32)])(
        edge_features, p['ee_w1'], p['ee_b1'].reshape(1, H),
        p['ee_w2'], p['ee_b2'].reshape(1, H),
        lp0['We'], lp0['a'][128:192, 0].reshape(1, H))

    for li in range(3):
        lp = p['layers'][li]
        aq, ak, hmsg = _tc(
            _node_pre_body, N // BN,
            [_rows((BN, H)), _full((H, H)), _full((H, H)), _full((H, H)),
             _full((1, H)), _full((1, H))],
            [_rows((BN, 1)), _rows((BN, 1)), _rows((BN, H))],
            [jax.ShapeDtypeStruct((N, 1), _f32),
             jax.ShapeDtypeStruct((N, 1), _f32),
             jax.ShapeDtypeStruct((N, H), _f32)])(
            h, lp['Wq'], lp['Wk'], lp['Wmsg'],
            lp['a'][0:64, 0].reshape(1, H), lp['a'][64:128, 0].reshape(1, H))

        betar = jnp.pad(beta[:, 0], (0, EPAD - E)).reshape(ER, 128)
        s_sum, agg_raw = _sc_attn_agg(srcr, dstr, betar, ak[:, 0], hmsg)

        hn, ta, tb = _tc(
            _node_post_body, N // BN,
            [_rows((BN, H)), _rows((BN, H)), _rows((BN, 1)), _rows((BN, 1)),
             _full((H, H)), _full((1, H)), _full((1, H)), _full((1, H)),
             _full((H, H)), _full((H, H))],
            [_rows((BN, H)), _rows((BN, H)), _rows((BN, H))],
            [jax.ShapeDtypeStruct((N, H), _f32),
             jax.ShapeDtypeStruct((N, H), _f32),
             jax.ShapeDtypeStruct((N, H), _f32)])(
            h, agg_raw, s_sum.reshape(N, 1), aq, lp['Wself'],
            lp['bself'].reshape(1, H), lp['ln_g'].reshape(1, H),
            lp['ln_b'].reshape(1, H), lp['Weu'][0:64], lp['Weu'][64:128])

        ga, gb = _sc_pair_gather(
            ta, jnp.pad(tb, ((0, NPAD - N), (0, 0))), srcr, dstr)

        if li < 2:
            lpn = p['layers'][li + 1]
            e, beta = _tc(
                _edge_post_beta_body, E // BE,
                [_rows((BE, H)), _rows((BE, H)), _rows((BE, H)), _full((H, H)),
                 _full((1, H)), _full((H, H)), _full((1, H))],
                [_rows((BE, H)), _rows((BE, 1))],
                [jax.ShapeDtypeStruct((E, H), _f32),
                 jax.ShapeDtypeStruct((E, 1), _f32)])(
                e, ga, gb, lp['Weu'][128:192], lp['beu'].reshape(1, H),
                lpn['We'], lpn['a'][128:192, 0].reshape(1, H))
        else:
            e = _tc(
                _edge_post_body, E // BE,
                [_rows((BE, H)), _rows((BE, H)), _rows((BE, H)), _full((H, H)),
                 _full((1, H))],
                _rows((BE, H)), jax.ShapeDtypeStruct((E, H), _f32))(
                e, ga, gb, lp['Weu'][128:192], lp['beu'].reshape(1, H))
        h = hn

    u, v = _tc(_node_final_body, N // BN,
               [_rows((BN, H)), _full((H, H)), _full((H, H))],
               [_rows((BN, H)), _rows((BN, H))],
               [jax.ShapeDtypeStruct((N, H), _f32),
                jax.ShapeDtypeStruct((N, H), _f32)])(
        h, p['em_w1'][0:64], p['em_w1'][64:128])

    gu, gv = _sc_pair_gather(u, jnp.pad(v, ((0, NPAD - N), (0, 0))), srcr, dstr)

    scores = _tc(
        _final_body, E // BE,
        [_rows((BE, H)), _rows((BE, H)), _rows((BE, H)), _full((H, H)),
         _full((1, H)), _full((1, H)), _full((1, 1))],
        _rows((BE, 1)), jax.ShapeDtypeStruct((E, 1), _f32))(
        e, gu, gv, p['em_w1'][128:192], p['em_b1'].reshape(1, H),
        p['em_w2'][:, 0].reshape(1, H), p['em_b2'].reshape(1, 1))

    return scores[:, 0]


# speedup vs baseline: 4.0192x; 1.2585x over previous
import functools

import jax
import jax.numpy as jnp
from jax import lax
from jax.experimental import pallas as pl
from jax.experimental.pallas import tpu as pltpu
from jax.experimental.pallas import tpu_sc as plsc

N = 50000
E = 800000
H = 64
NPAD = 50048
EPAD = 819200
ER = EPAD // 128
BN = 2000
BE = 8000
HALF = 25000
TBL = 25088
PREC = lax.Precision.HIGHEST

_f32 = jnp.float32
_i32 = jnp.int32


def _lr(x):
    return jnp.where(x >= 0, x, 0.2 * x)


def _rows(shape):
    nd = len(shape)
    return pl.BlockSpec(shape, lambda i: (i,) + (0,) * (nd - 1))


def _full(shape):
    nd = len(shape)
    return pl.BlockSpec(shape, lambda i: (0,) * nd)


def _dot(a, b):
    return jnp.dot(a, b, precision=PREC)


def _enc_body(x, w1, b1, w2, b2, o):
    o[...] = _dot(jax.nn.relu(_dot(x[...], w1[...]) + b1[...]), w2[...]) + b2[...]


def _enc_edges_body(x, w1, b1, w2, b2, we, ae, oe, ob):
    e = _dot(jax.nn.relu(_dot(x[...], w1[...]) + b1[...]), w2[...]) + b2[...]
    oe[...] = e
    ob[...] = jnp.sum(_lr(_dot(e, we[...])) * ae[...], -1, keepdims=True)


def _node_pre_body(h, wq, wk, wm, aqr, akr, oaq, oak, om):
    hh = h[...]
    oaq[...] = jnp.sum(_lr(_dot(hh, wq[...])) * aqr[...], -1, keepdims=True)
    oak[...] = jnp.sum(_lr(_dot(hh, wk[...])) * akr[...], -1, keepdims=True)
    om[...] = _dot(hh, wm[...])


def _node_post_body(h, agg, sr, aqc, wself, bself, g, b, wa, wb, ohn, oa, ob):
    hh = h[...]
    a = agg[...] / (sr[...] + 1e-12 * jnp.exp(-aqc[...]))
    hn = jax.nn.relu(_dot(hh, wself[...]) + bself[...] + a)
    x = hn + hh
    mu = jnp.mean(x, -1, keepdims=True)
    xc = x - mu
    var = jnp.mean(xc * xc, -1, keepdims=True)
    hn2 = xc / jnp.sqrt(var + 1e-5) * g[...] + b[...]
    ohn[...] = hn2
    oa[...] = _dot(hn2, wa[...])
    ob[...] = _dot(hn2, wb[...])


def _edge_post_beta_body(e, g, wee, beu, wen, aen, oe, ob):
    gg = g[...]
    en = jax.nn.relu(gg[:, 0:H] + gg[:, H:2 * H]
                     + _dot(e[...], wee[...]) + beu[...]) + e[...]
    oe[...] = en
    ob[...] = jnp.sum(_lr(_dot(en, wen[...])) * aen[...], -1, keepdims=True)


def _edge_post_body(e, g, wee, beu, oe):
    gg = g[...]
    oe[...] = jax.nn.relu(gg[:, 0:H] + gg[:, H:2 * H]
                          + _dot(e[...], wee[...]) + beu[...]) + e[...]


def _node_final_body(h, ws, wd, ou, ov):
    hh = h[...]
    ou[...] = _dot(hh, ws[...])
    ov[...] = _dot(hh, wd[...])


def _final_body(e, g, w1e, b1, w2r, b2, os):
    gg = g[...]
    hid = jax.nn.relu(gg[:, 0:H] + gg[:, H:2 * H]
                      + _dot(e[...], w1e[...]) + b1[...])
    os[...] = jax.nn.sigmoid(jnp.sum(hid * w2r[...], -1, keepdims=True) + b2[...])


def _tc(body, grid, in_specs, out_specs, out_shape):
    return pl.pallas_call(
        body, grid=(grid,), in_specs=in_specs, out_specs=out_specs,
        out_shape=out_shape)


_MESH = plsc.VectorSubcoreMesh(core_axis_name="c", subcore_axis_name="s")
_SC_PARAMS = pltpu.CompilerParams(
    needs_layout_passes=False, use_tc_tiling_on_sc=False)


def _sc_pair_gather_body(ta, tb, sdr, g, sd, ra0, rb0, ra1, rb1,
                         gs0, gs1, ws0, ws1):
    c = lax.axis_index("c")
    s = lax.axis_index("s")
    wid = s * 2 + c
    wbase = wid * 200

    def fire_g(half, ra, rb, gs):
        for j in range(2):
            pltpu.async_copy(ta.at[sd.at[half * 2 + j, 0]],
                             ra.at[pl.ds(j * 128, 128)], gs)
            pltpu.async_copy(tb.at[sd.at[half * 2 + j, 1]],
                             rb.at[pl.ds(j * 128, 128)], gs)

    def wait_g(ra, rb, gs):
        pltpu.make_async_copy(ta.at[pl.ds(0, 256)], ra, gs).wait()
        pltpu.make_async_copy(tb.at[pl.ds(0, 256)], rb, gs).wait()

    def fire_w(eb, ra, rb, ws):
        pltpu.async_copy(ra, g.at[pl.ds(eb, 256), pl.ds(0, H)], ws)
        pltpu.async_copy(rb, g.at[pl.ds(eb, 256), pl.ds(H, H)], ws)

    def wait_w(ra, rb, ws):
        pltpu.make_async_copy(ra, g.at[pl.ds(0, 256), pl.ds(0, H)], ws).wait()
        pltpu.make_async_copy(rb, g.at[pl.ds(0, 256), pl.ds(H, H)], ws).wait()

    pltpu.sync_copy(sdr.at[pl.ds(wbase, 4)], sd)
    fire_g(0, ra0, rb0, gs0)
    fire_g(1, ra1, rb1, gs1)

    def pair(p, carry):
        row0 = wbase + p * 4
        eb0 = pl.multiple_of(row0 * 128, 256)
        eb1 = pl.multiple_of((row0 + 2) * 128, 256)
        wait_g(ra0, rb0, gs0)
        fire_w(eb0, ra0, rb0, ws0)
        wait_g(ra1, rb1, gs1)
        fire_w(eb1, ra1, rb1, ws1)

        @pl.when(p < 49)
        def _pref():
            pltpu.sync_copy(sdr.at[pl.ds(row0 + 4, 4)], sd)
            wait_w(ra0, rb0, ws0)
            fire_g(0, ra0, rb0, gs0)
            wait_w(ra1, rb1, ws1)
            fire_g(1, ra1, rb1, gs1)

        return carry

    lax.fori_loop(0, 50, pair, 0)
    wait_w(ra0, rb0, ws0)
    wait_w(ra1, rb1, ws1)


_sc_pair_gather = pl.kernel(
    _sc_pair_gather_body,
    out_type=jax.ShapeDtypeStruct((EPAD, 2 * H), _f32),
    mesh=_MESH,
    compiler_params=_SC_PARAMS,
    scratch_types=[
        pltpu.VMEM((4, 2, 128), _i32),
        pltpu.VMEM((256, H), _f32),
        pltpu.VMEM((256, H), _f32),
        pltpu.VMEM((256, H), _f32),
        pltpu.VMEM((256, H), _f32),
        pltpu.SemaphoreType.DMA,
        pltpu.SemaphoreType.DMA,
        pltpu.SemaphoreType.DMA,
        pltpu.SemaphoreType.DMA,
    ],
)


def _make_sc_attn_body():
    def body(sdr, akt, hmsg, s_out, agg_out,
             sd, akv0, akv1, sv0, sv1, sf0, sf1, li0, li1, rows0, rows1, zs,
             s_sh, agg_sh, gs0, gs1, ss0, ss1):
        c = lax.axis_index("c")
        s = lax.axis_index("s")
        base = c * HALF
        sbase = s * 400
        zero16 = jnp.zeros((16,), _f32)
        iota16 = lax.iota(_i32, 16)

        def zrow(i, carry):
            for jj in range(4):
                rows0[i, pl.ds(jj * 16, 16)] = zero16
            return carry

        lax.fori_loop(0, 128, zrow, 0)

        def zz(i, carry):
            zs[pl.ds(i * 16, 16)] = zero16
            return carry

        lax.fori_loop(0, 98, zz, 0)

        r0 = pl.multiple_of(s * 1568, 8)
        for t in range(12):
            pltpu.sync_copy(rows0, agg_sh.at[pl.ds(r0 + t * 128, 128)])
        pltpu.sync_copy(rows0.at[pl.ds(0, 32)], agg_sh.at[pl.ds(r0 + 1536, 32)])
        pltpu.sync_copy(zs, s_sh.at[pl.ds(r0, 1568)])
        plsc.subcore_barrier()

        def fire_g(half, akv, rows, gs):
            pltpu.async_copy(akt.at[sd.at[half, 0]], akv.at[0], gs)
            pltpu.async_copy(hmsg.at[sd.at[half, 0]], rows, gs)

        def wait_g(akv, rows, gs):
            pltpu.make_async_copy(akt.at[pl.ds(0, 128)], akv.at[0], gs).wait()
            pltpu.make_async_copy(hmsg.at[pl.ds(0, 128)], rows, gs).wait()

        def fire_s(akv, rows, sv, li, ss):
            pltpu.async_copy(rows, agg_sh.at[li.at[0]], ss, add=True)
            pltpu.async_copy(sv.at[0], s_sh.at[li.at[0]], ss, add=True)

        def wait_s(rows, sv, li, ss):
            pltpu.make_async_copy(rows, agg_sh.at[li.at[0]], ss).wait()
            pltpu.make_async_copy(sv.at[0], s_sh.at[li.at[0]], ss).wait()

        def compute(half, akv, rows, sv, sf, li):
            def grp(i, carry):
                dsl = pl.ds(i * 16, 16)
                bv = plsc.bitcast(sd[half, 2, dsl], _f32)
                sval = jnp.exp(akv[0, dsl] + bv)
                sv[0, dsl] = sval
                sf[pl.ds(i * 16, 16)] = sval
                d = sd[half, 1, dsl] - base
                owned = (d >= 0) & (d < HALF)
                tr = HALF + iota16 + (i % 4) * 16
                li[0, dsl] = jnp.where(owned, d, tr)
                return carry

            lax.fori_loop(0, 8, grp, 0)

            def scale(ei, carry):
                spl = plsc.load_gather(sf, [jnp.full((16,), ei, _i32)])
                for jj in range(4):
                    dsl = pl.ds(jj * 16, 16)
                    rows[ei, dsl] = rows[ei, dsl] * spl
                return carry

            lax.fori_loop(0, 128, scale, 0)

        pltpu.sync_copy(sdr.at[pl.ds(sbase, 2)], sd)
        fire_g(0, akv0, rows0, gs0)
        fire_g(1, akv1, rows1, gs1)

        def pair(p, carry):
            row0 = sbase + p * 2
            wait_g(akv0, rows0, gs0)
            compute(0, akv0, rows0, sv0, sf0, li0)
            fire_s(akv0, rows0, sv0, li0, ss0)
            wait_g(akv1, rows1, gs1)
            compute(1, akv1, rows1, sv1, sf1, li1)
            fire_s(akv1, rows1, sv1, li1, ss1)

            @pl.when(p < 199)
            def _pref():
                pltpu.sync_copy(sdr.at[pl.ds(row0 + 2, 2)], sd)
                wait_s(rows0, sv0, li0, ss0)
                fire_g(0, akv0, rows0, gs0)
                wait_s(rows1, sv1, li1, ss1)
                fire_g(1, akv1, rows1, gs1)

            return carry

        lax.fori_loop(0, 200, pair, 0)
        wait_s(rows0, sv0, li0, ss0)
        wait_s(rows1, sv1, li1, ss1)
        plsc.subcore_barrier()

        o0 = pl.multiple_of(s * 1560, 8)
        pltpu.sync_copy(agg_sh.at[pl.ds(o0, 1560)],
                        agg_out.at[pl.ds(base + o0, 1560)])
        pltpu.sync_copy(s_sh.at[pl.ds(o0, 1560)],
                        s_out.at[pl.ds(base + o0, 1560)])

        @pl.when(s == 0)
        def _tail():
            pltpu.sync_copy(agg_sh.at[pl.ds(24960, 40)],
                            agg_out.at[pl.ds(base + 24960, 40)])
            pltpu.sync_copy(s_sh.at[pl.ds(24960, 40)],
                            s_out.at[pl.ds(base + 24960, 40)])

    return body


_sc_attn_agg = pl.kernel(
    _make_sc_attn_body(),
    out_type=[jax.ShapeDtypeStruct((N,), _f32),
              jax.ShapeDtypeStruct((N, H), _f32)],
    mesh=_MESH,
    compiler_params=_SC_PARAMS,
    scratch_types=[
        pltpu.VMEM((2, 3, 128), _i32),
        pltpu.VMEM((1, 128), _f32),
        pltpu.VMEM((1, 128), _f32),
        pltpu.VMEM((1, 128), _f32),
        pltpu.VMEM((1, 128), _f32),
        pltpu.VMEM((128,), _f32),
        pltpu.VMEM((128,), _f32),
        pltpu.VMEM((1, 128), _i32),
        pltpu.VMEM((1, 128), _i32),
        pltpu.VMEM((128, H), _f32),
        pltpu.VMEM((128, H), _f32),
        pltpu.VMEM((1568,), _f32),
        pltpu.VMEM_SHARED((TBL,), _f32),
        pltpu.VMEM_SHARED((TBL, H), _f32),
        pltpu.SemaphoreType.DMA,
        pltpu.SemaphoreType.DMA,
        pltpu.SemaphoreType.DMA,
        pltpu.SemaphoreType.DMA,
    ],
)


def kernel(node_features, edge_features, edge_index, params):
    p = params
    src = edge_index[0].astype(_i32)
    dst = edge_index[1].astype(_i32)
    srcr = jnp.pad(src, (0, EPAD - E)).reshape(ER, 128)
    dstr = jnp.pad(dst, (0, EPAD - E), constant_values=N).reshape(ER, 128)
    sdr2 = jnp.stack([srcr, dstr], axis=1)

    h = _tc(_enc_body, N // BN,
            [_rows((BN, 8)), _full((8, H)), _full((1, H)), _full((H, H)),
             _full((1, H))],
            _rows((BN, H)), jax.ShapeDtypeStruct((N, H), _f32))(
        node_features, p['ne_w1'], p['ne_b1'].reshape(1, H),
        p['ne_w2'], p['ne_b2'].reshape(1, H))

    lp0 = p['layers'][0]
    e, beta = _tc(_enc_edges_body, E // BE,
                  [_rows((BE, 8)), _full((8, H)), _full((1, H)), _full((H, H)),
                   _full((1, H)), _full((H, H)), _full((1, H))],
                  [_rows((BE, H)), _rows((BE, 1))],
                  [jax.ShapeDtypeStruct((E, H), _f32),
                   jax.ShapeDtypeStruct((E, 1), _f32)])(
        edge_features, p['ee_w1'], p['ee_b1'].reshape(1, H),
        p['ee_w2'], p['ee_b2'].reshape(1, H),
        lp0['We'], lp0['a'][128:192, 0].reshape(1, H))

    for li in range(3):
        lp = p['layers'][li]
        aq, ak, hmsg = _tc(
            _node_pre_body, N // BN,
            [_rows((BN, H)), _full((H, H)), _full((H, H)), _full((H, H)),
             _full((1, H)), _full((1, H))],
            [_rows((BN, 1)), _rows((BN, 1)), _rows((BN, H))],
            [jax.ShapeDtypeStruct((N, 1), _f32),
             jax.ShapeDtypeStruct((N, 1), _f32),
             jax.ShapeDtypeStruct((N, H), _f32)])(
            h, lp['Wq'], lp['Wk'], lp['Wmsg'],
            lp['a'][0:64, 0].reshape(1, H), lp['a'][64:128, 0].reshape(1, H))

        betar = lax.bitcast_convert_type(
            jnp.pad(beta[:, 0], (0, EPAD - E)).reshape(ER, 128), _i32)
        sdr3 = jnp.stack([srcr, dstr, betar], axis=1)
        s_sum, agg_raw = _sc_attn_agg(sdr3, ak[:, 0], hmsg)

        hn, ta, tb = _tc(
            _node_post_body, N // BN,
            [_rows((BN, H)), _rows((BN, H)), _rows((BN, 1)), _rows((BN, 1)),
             _full((H, H)), _full((1, H)), _full((1, H)), _full((1, H)),
             _full((H, H)), _full((H, H))],
            [_rows((BN, H)), _rows((BN, H)), _rows((BN, H))],
            [jax.ShapeDtypeStruct((N, H), _f32),
             jax.ShapeDtypeStruct((N, H), _f32),
             jax.ShapeDtypeStruct((N, H), _f32)])(
            h, agg_raw, s_sum.reshape(N, 1), aq, lp['Wself'],
            lp['bself'].reshape(1, H), lp['ln_g'].reshape(1, H),
            lp['ln_b'].reshape(1, H), lp['Weu'][0:64], lp['Weu'][64:128])

        g = _sc_pair_gather(
            ta, jnp.pad(tb, ((0, NPAD - N), (0, 0))), sdr2)

        if li < 2:
            lpn = p['layers'][li + 1]
            e, beta = _tc(
                _edge_post_beta_body, E // BE,
                [_rows((BE, H)), _rows((BE, 2 * H)), _full((H, H)),
                 _full((1, H)), _full((H, H)), _full((1, H))],
                [_rows((BE, H)), _rows((BE, 1))],
                [jax.ShapeDtypeStruct((E, H), _f32),
                 jax.ShapeDtypeStruct((E, 1), _f32)])(
                e, g, lp['Weu'][128:192], lp['beu'].reshape(1, H),
                lpn['We'], lpn['a'][128:192, 0].reshape(1, H))
        else:
            e = _tc(
                _edge_post_body, E // BE,
                [_rows((BE, H)), _rows((BE, 2 * H)), _full((H, H)),
                 _full((1, H))],
                _rows((BE, H)), jax.ShapeDtypeStruct((E, H), _f32))(
                e, g, lp['Weu'][128:192], lp['beu'].reshape(1, H))
        h = hn

    u, v = _tc(_node_final_body, N // BN,
               [_rows((BN, H)), _full((H, H)), _full((H, H))],
               [_rows((BN, H)), _rows((BN, H))],
               [jax.ShapeDtypeStruct((N, H), _f32),
                jax.ShapeDtypeStruct((N, H), _f32)])(
        h, p['em_w1'][0:64], p['em_w1'][64:128])

    gf = _sc_pair_gather(u, jnp.pad(v, ((0, NPAD - N), (0, 0))), sdr2)

    scores = _tc(
        _final_body, E // BE,
        [_rows((BE, H)), _rows((BE, 2 * H)), _full((H, H)),
         _full((1, H)), _full((1, H)), _full((1, 1))],
        _rows((BE, 1)), jax.ShapeDtypeStruct((E, 1), _f32))(
        e, gf, p['em_w1'][128:192], p['em_b1'].reshape(1, H),
        p['em_w2'][:, 0].reshape(1, H), p['em_b2'].reshape(1, 1))

    return scores[:, 0]


# speedup vs baseline: 5.0539x; 1.2574x over previous
import functools

import jax
import jax.numpy as jnp
from jax import lax
from jax.experimental import pallas as pl
from jax.experimental.pallas import tpu as pltpu
from jax.experimental.pallas import tpu_sc as plsc

N = 50000
E = 800000
H = 64
NPAD = 50048
EPAD = 819200
ER = EPAD // 128
BN = 2000
BE = 8000
HALF = 25000
TBL = 25088
PREC = lax.Precision.DEFAULT

_f32 = jnp.float32
_i32 = jnp.int32


def _lr(x):
    return jnp.where(x >= 0, x, 0.2 * x)


def _rows(shape):
    nd = len(shape)
    return pl.BlockSpec(shape, lambda i: (i,) + (0,) * (nd - 1))


def _full(shape):
    nd = len(shape)
    return pl.BlockSpec(shape, lambda i: (0,) * nd)


def _dot(a, b):
    return jnp.dot(a, b, precision=PREC)


def _enc_body(x, w1, b1, w2, b2, o):
    o[...] = _dot(jax.nn.relu(_dot(x[...], w1[...]) + b1[...]), w2[...]) + b2[...]


def _enc_edges_body(x, w1, b1, w2, b2, we, ae, oe, ob):
    e = _dot(jax.nn.relu(_dot(x[...], w1[...]) + b1[...]), w2[...]) + b2[...]
    oe[...] = e
    ob[...] = jnp.sum(_lr(_dot(e, we[...])) * ae[...], -1, keepdims=True)


def _node_pre_body(h, wq, wk, wm, aqr, akr, oaq, oak, om):
    hh = h[...]
    oaq[...] = jnp.sum(_lr(_dot(hh, wq[...])) * aqr[...], -1, keepdims=True)
    oak[...] = jnp.sum(_lr(_dot(hh, wk[...])) * akr[...], -1, keepdims=True)
    om[...] = _dot(hh, wm[...])


def _node_post_body(h, agg, sr, aqc, wself, bself, g, b, wa, wb, ohn, oa, ob):
    hh = h[...]
    a = agg[...] / (sr[...] + 1e-12 * jnp.exp(-aqc[...]))
    hn = jax.nn.relu(_dot(hh, wself[...]) + bself[...] + a)
    x = hn + hh
    mu = jnp.mean(x, -1, keepdims=True)
    xc = x - mu
    var = jnp.mean(xc * xc, -1, keepdims=True)
    hn2 = xc / jnp.sqrt(var + 1e-5) * g[...] + b[...]
    ohn[...] = hn2
    oa[...] = _dot(hn2, wa[...])
    ob[...] = _dot(hn2, wb[...])


def _edge_post_beta_body(e, g, wee, beu, wen, aen, oe, ob):
    gg = g[...]
    en = jax.nn.relu(gg[:, 0:H] + gg[:, H:2 * H]
                     + _dot(e[...], wee[...]) + beu[...]) + e[...]
    oe[...] = en
    ob[...] = jnp.sum(_lr(_dot(en, wen[...])) * aen[...], -1, keepdims=True)


def _edge_post_body(e, g, wee, beu, oe):
    gg = g[...]
    oe[...] = jax.nn.relu(gg[:, 0:H] + gg[:, H:2 * H]
                          + _dot(e[...], wee[...]) + beu[...]) + e[...]


def _node_final_body(h, ws, wd, ou, ov):
    hh = h[...]
    ou[...] = _dot(hh, ws[...])
    ov[...] = _dot(hh, wd[...])


def _final_body(e, g, w1e, b1, w2r, b2, os):
    gg = g[...]
    hid = jax.nn.relu(gg[:, 0:H] + gg[:, H:2 * H]
                      + _dot(e[...], w1e[...]) + b1[...])
    os[...] = jax.nn.sigmoid(jnp.sum(hid * w2r[...], -1, keepdims=True) + b2[...])


def _tc(body, grid, in_specs, out_specs, out_shape):
    return pl.pallas_call(
        body, grid=(grid,), in_specs=in_specs, out_specs=out_specs,
        out_shape=out_shape)


_MESH = plsc.VectorSubcoreMesh(core_axis_name="c", subcore_axis_name="s")
_SC_PARAMS = pltpu.CompilerParams(
    needs_layout_passes=False, use_tc_tiling_on_sc=False)


def _sc_pair_gather_body(ta, tb, sdr, g, sd, ra0, rb0, ra1, rb1,
                         gs0, gs1, ws0, ws1):
    c = lax.axis_index("c")
    s = lax.axis_index("s")
    wid = s * 2 + c
    wbase = wid * 200

    def fire_g(half, ra, rb, gs):
        for j in range(2):
            pltpu.async_copy(ta.at[sd.at[half * 2 + j, 0]],
                             ra.at[pl.ds(j * 128, 128)], gs)
            pltpu.async_copy(tb.at[sd.at[half * 2 + j, 1]],
                             rb.at[pl.ds(j * 128, 128)], gs)

    def wait_g(ra, rb, gs):
        pltpu.make_async_copy(ta.at[pl.ds(0, 256)], ra, gs).wait()
        pltpu.make_async_copy(tb.at[pl.ds(0, 256)], rb, gs).wait()

    def fire_w(eb, ra, rb, ws):
        pltpu.async_copy(ra, g.at[pl.ds(eb, 256), pl.ds(0, H)], ws)
        pltpu.async_copy(rb, g.at[pl.ds(eb, 256), pl.ds(H, H)], ws)

    def wait_w(ra, rb, ws):
        pltpu.make_async_copy(ra, g.at[pl.ds(0, 256), pl.ds(0, H)], ws).wait()
        pltpu.make_async_copy(rb, g.at[pl.ds(0, 256), pl.ds(H, H)], ws).wait()

    pltpu.sync_copy(sdr.at[pl.ds(wbase, 4)], sd)
    fire_g(0, ra0, rb0, gs0)
    fire_g(1, ra1, rb1, gs1)

    def pair(p, carry):
        row0 = wbase + p * 4
        eb0 = pl.multiple_of(row0 * 128, 256)
        eb1 = pl.multiple_of((row0 + 2) * 128, 256)
        wait_g(ra0, rb0, gs0)
        fire_w(eb0, ra0, rb0, ws0)
        wait_g(ra1, rb1, gs1)
        fire_w(eb1, ra1, rb1, ws1)

        @pl.when(p < 49)
        def _pref():
            pltpu.sync_copy(sdr.at[pl.ds(row0 + 4, 4)], sd)
            wait_w(ra0, rb0, ws0)
            fire_g(0, ra0, rb0, gs0)
            wait_w(ra1, rb1, ws1)
            fire_g(1, ra1, rb1, gs1)

        return carry

    lax.fori_loop(0, 50, pair, 0)
    wait_w(ra0, rb0, ws0)
    wait_w(ra1, rb1, ws1)


_sc_pair_gather = pl.kernel(
    _sc_pair_gather_body,
    out_type=jax.ShapeDtypeStruct((EPAD, 2 * H), _f32),
    mesh=_MESH,
    compiler_params=_SC_PARAMS,
    scratch_types=[
        pltpu.VMEM((4, 2, 128), _i32),
        pltpu.VMEM((256, H), _f32),
        pltpu.VMEM((256, H), _f32),
        pltpu.VMEM((256, H), _f32),
        pltpu.VMEM((256, H), _f32),
        pltpu.SemaphoreType.DMA,
        pltpu.SemaphoreType.DMA,
        pltpu.SemaphoreType.DMA,
        pltpu.SemaphoreType.DMA,
    ],
)


def _make_sc_attn_body():
    def body(sdr, akt, hmsg, s_out, agg_out,
             sd, akv0, akv1, sv0, sv1, sf0, sf1, li0, li1, rows0, rows1, zs,
             s_sh, agg_sh, gs0, gs1, ss0, ss1):
        c = lax.axis_index("c")
        s = lax.axis_index("s")
        base = c * HALF
        sbase = s * 400
        zero16 = jnp.zeros((16,), _f32)
        iota16 = lax.iota(_i32, 16)

        def zrow(i, carry):
            for jj in range(4):
                rows0[i, pl.ds(jj * 16, 16)] = zero16
            return carry

        lax.fori_loop(0, 128, zrow, 0)

        def zz(i, carry):
            zs[pl.ds(i * 16, 16)] = zero16
            return carry

        lax.fori_loop(0, 98, zz, 0)

        r0 = pl.multiple_of(s * 1568, 8)
        for t in range(12):
            pltpu.sync_copy(rows0, agg_sh.at[pl.ds(r0 + t * 128, 128)])
        pltpu.sync_copy(rows0.at[pl.ds(0, 32)], agg_sh.at[pl.ds(r0 + 1536, 32)])
        pltpu.sync_copy(zs, s_sh.at[pl.ds(r0, 1568)])
        plsc.subcore_barrier()

        def fire_g(half, akv, rows, gs):
            pltpu.async_copy(akt.at[sd.at[half, 0]], akv.at[0], gs)
            pltpu.async_copy(hmsg.at[sd.at[half, 0]], rows, gs)

        def wait_g(akv, rows, gs):
            pltpu.make_async_copy(akt.at[pl.ds(0, 128)], akv.at[0], gs).wait()
            pltpu.make_async_copy(hmsg.at[pl.ds(0, 128)], rows, gs).wait()

        def fire_s(akv, rows, sv, li, ss):
            pltpu.async_copy(rows, agg_sh.at[li.at[0]], ss, add=True)
            pltpu.async_copy(sv.at[0], s_sh.at[li.at[0]], ss, add=True)

        def wait_s(rows, sv, li, ss):
            pltpu.make_async_copy(rows, agg_sh.at[li.at[0]], ss).wait()
            pltpu.make_async_copy(sv.at[0], s_sh.at[li.at[0]], ss).wait()

        def compute(half, akv, rows, sv, sf, li):
            def grp(i, carry):
                dsl = pl.ds(i * 16, 16)
                bv = plsc.bitcast(sd[half, 2, dsl], _f32)
                sval = jnp.exp(akv[0, dsl] + bv)
                sv[0, dsl] = sval
                sf[pl.ds(i * 16, 16)] = sval
                d = sd[half, 1, dsl] - base
                owned = (d >= 0) & (d < HALF)
                tr = HALF + iota16 + (i % 4) * 16
                li[0, dsl] = jnp.where(owned, d, tr)
                return carry

            lax.fori_loop(0, 8, grp, 0)

            def scale(ei, carry):
                spl = plsc.load_gather(sf, [jnp.full((16,), ei, _i32)])
                for jj in range(4):
                    dsl = pl.ds(jj * 16, 16)
                    rows[ei, dsl] = rows[ei, dsl] * spl
                return carry

            lax.fori_loop(0, 128, scale, 0)

        pltpu.sync_copy(sdr.at[pl.ds(sbase, 2)], sd)
        fire_g(0, akv0, rows0, gs0)
        fire_g(1, akv1, rows1, gs1)

        def pair(p, carry):
            row0 = sbase + p * 2
            wait_g(akv0, rows0, gs0)
            compute(0, akv0, rows0, sv0, sf0, li0)
            fire_s(akv0, rows0, sv0, li0, ss0)
            wait_g(akv1, rows1, gs1)
            compute(1, akv1, rows1, sv1, sf1, li1)
            fire_s(akv1, rows1, sv1, li1, ss1)

            @pl.when(p < 199)
            def _pref():
                pltpu.sync_copy(sdr.at[pl.ds(row0 + 2, 2)], sd)
                wait_s(rows0, sv0, li0, ss0)
                fire_g(0, akv0, rows0, gs0)
                wait_s(rows1, sv1, li1, ss1)
                fire_g(1, akv1, rows1, gs1)

            return carry

        lax.fori_loop(0, 200, pair, 0)
        wait_s(rows0, sv0, li0, ss0)
        wait_s(rows1, sv1, li1, ss1)
        plsc.subcore_barrier()

        o0 = pl.multiple_of(s * 1560, 8)
        pltpu.sync_copy(agg_sh.at[pl.ds(o0, 1560)],
                        agg_out.at[pl.ds(base + o0, 1560)])
        pltpu.sync_copy(s_sh.at[pl.ds(o0, 1560)],
                        s_out.at[pl.ds(base + o0, 1560)])

        @pl.when(s == 0)
        def _tail():
            pltpu.sync_copy(agg_sh.at[pl.ds(24960, 40)],
                            agg_out.at[pl.ds(base + 24960, 40)])
            pltpu.sync_copy(s_sh.at[pl.ds(24960, 40)],
                            s_out.at[pl.ds(base + 24960, 40)])

    return body


_sc_attn_agg = pl.kernel(
    _make_sc_attn_body(),
    out_type=[jax.ShapeDtypeStruct((N,), _f32),
              jax.ShapeDtypeStruct((N, H), _f32)],
    mesh=_MESH,
    compiler_params=_SC_PARAMS,
    scratch_types=[
        pltpu.VMEM((2, 3, 128), _i32),
        pltpu.VMEM((1, 128), _f32),
        pltpu.VMEM((1, 128), _f32),
        pltpu.VMEM((1, 128), _f32),
        pltpu.VMEM((1, 128), _f32),
        pltpu.VMEM((128,), _f32),
        pltpu.VMEM((128,), _f32),
        pltpu.VMEM((1, 128), _i32),
        pltpu.VMEM((1, 128), _i32),
        pltpu.VMEM((128, H), _f32),
        pltpu.VMEM((128, H), _f32),
        pltpu.VMEM((1568,), _f32),
        pltpu.VMEM_SHARED((TBL,), _f32),
        pltpu.VMEM_SHARED((TBL, H), _f32),
        pltpu.SemaphoreType.DMA,
        pltpu.SemaphoreType.DMA,
        pltpu.SemaphoreType.DMA,
        pltpu.SemaphoreType.DMA,
    ],
)


def kernel(node_features, edge_features, edge_index, params):
    p = params
    src = edge_index[0].astype(_i32)
    dst = edge_index[1].astype(_i32)
    srcr = jnp.pad(src, (0, EPAD - E)).reshape(ER, 128)
    dstr = jnp.pad(dst, (0, EPAD - E), constant_values=N).reshape(ER, 128)
    sdr2 = jnp.stack([srcr, dstr], axis=1)

    h = _tc(_enc_body, N // BN,
            [_rows((BN, 8)), _full((8, H)), _full((1, H)), _full((H, H)),
             _full((1, H))],
            _rows((BN, H)), jax.ShapeDtypeStruct((N, H), _f32))(
        node_features, p['ne_w1'], p['ne_b1'].reshape(1, H),
        p['ne_w2'], p['ne_b2'].reshape(1, H))

    lp0 = p['layers'][0]
    e, beta = _tc(_enc_edges_body, E // BE,
                  [_rows((BE, 8)), _full((8, H)), _full((1, H)), _full((H, H)),
                   _full((1, H)), _full((H, H)), _full((1, H))],
                  [_rows((BE, H)), _rows((BE, 1))],
                  [jax.ShapeDtypeStruct((E, H), _f32),
                   jax.ShapeDtypeStruct((E, 1), _f32)])(
        edge_features, p['ee_w1'], p['ee_b1'].reshape(1, H),
        p['ee_w2'], p['ee_b2'].reshape(1, H),
        lp0['We'], lp0['a'][128:192, 0].reshape(1, H))

    for li in range(3):
        lp = p['layers'][li]
        aq, ak, hmsg = _tc(
            _node_pre_body, N // BN,
            [_rows((BN, H)), _full((H, H)), _full((H, H)), _full((H, H)),
             _full((1, H)), _full((1, H))],
            [_rows((BN, 1)), _rows((BN, 1)), _rows((BN, H))],
            [jax.ShapeDtypeStruct((N, 1), _f32),
             jax.ShapeDtypeStruct((N, 1), _f32),
             jax.ShapeDtypeStruct((N, H), _f32)])(
            h, lp['Wq'], lp['Wk'], lp['Wmsg'],
            lp['a'][0:64, 0].reshape(1, H), lp['a'][64:128, 0].reshape(1, H))

        betar = lax.bitcast_convert_type(
            jnp.pad(beta[:, 0], (0, EPAD - E)).reshape(ER, 128), _i32)
        sdr3 = jnp.stack([srcr, dstr, betar], axis=1)
        s_sum, agg_raw = _sc_attn_agg(sdr3, ak[:, 0], hmsg)

        hn, ta, tb = _tc(
            _node_post_body, N // BN,
            [_rows((BN, H)), _rows((BN, H)), _rows((BN, 1)), _rows((BN, 1)),
             _full((H, H)), _full((1, H)), _full((1, H)), _full((1, H)),
             _full((H, H)), _full((H, H))],
            [_rows((BN, H)), _rows((BN, H)), _rows((BN, H))],
            [jax.ShapeDtypeStruct((N, H), _f32),
             jax.ShapeDtypeStruct((N, H), _f32),
             jax.ShapeDtypeStruct((N, H), _f32)])(
            h, agg_raw, s_sum.reshape(N, 1), aq, lp['Wself'],
            lp['bself'].reshape(1, H), lp['ln_g'].reshape(1, H),
            lp['ln_b'].reshape(1, H), lp['Weu'][0:64], lp['Weu'][64:128])

        g = _sc_pair_gather(
            ta, jnp.pad(tb, ((0, NPAD - N), (0, 0))), sdr2)

        if li < 2:
            lpn = p['layers'][li + 1]
            e, beta = _tc(
                _edge_post_beta_body, E // BE,
                [_rows((BE, H)), _rows((BE, 2 * H)), _full((H, H)),
                 _full((1, H)), _full((H, H)), _full((1, H))],
                [_rows((BE, H)), _rows((BE, 1))],
                [jax.ShapeDtypeStruct((E, H), _f32),
                 jax.ShapeDtypeStruct((E, 1), _f32)])(
                e, g, lp['Weu'][128:192], lp['beu'].reshape(1, H),
                lpn['We'], lpn['a'][128:192, 0].reshape(1, H))
        else:
            e = _tc(
                _edge_post_body, E // BE,
                [_rows((BE, H)), _rows((BE, 2 * H)), _full((H, H)),
                 _full((1, H))],
                _rows((BE, H)), jax.ShapeDtypeStruct((E, H), _f32))(
                e, g, lp['Weu'][128:192], lp['beu'].reshape(1, H))
        h = hn

    u, v = _tc(_node_final_body, N // BN,
               [_rows((BN, H)), _full((H, H)), _full((H, H))],
               [_rows((BN, H)), _rows((BN, H))],
               [jax.ShapeDtypeStruct((N, H), _f32),
                jax.ShapeDtypeStruct((N, H), _f32)])(
        h, p['em_w1'][0:64], p['em_w1'][64:128])

    gf = _sc_pair_gather(u, jnp.pad(v, ((0, NPAD - N), (0, 0))), sdr2)

    scores = _tc(
        _final_body, E // BE,
        [_rows((BE, H)), _rows((BE, 2 * H)), _full((H, H)),
         _full((1, H)), _full((1, H)), _full((1, 1))],
        _rows((BE, 1)), jax.ShapeDtypeStruct((E, 1), _f32))(
        e, gf, p['em_w1'][128:192], p['em_b1'].reshape(1, H),
        p['em_w2'][:, 0].reshape(1, H), p['em_b2'].reshape(1, 1))

    return scores[:, 0]


# speedup vs baseline: 5.0976x; 1.0086x over previous
import functools

import jax
import jax.numpy as jnp
from jax import lax
from jax.experimental import pallas as pl
from jax.experimental.pallas import tpu as pltpu
from jax.experimental.pallas import tpu_sc as plsc

N = 50000
E = 800000
H = 64
NPAD = 50048
EPAD = 819200
ER = EPAD // 128
BN = 2000
BE = 8000
HALF = 25000
TBL = 25088
PREC = lax.Precision.DEFAULT

_f32 = jnp.float32
_i32 = jnp.int32


def _lr(x):
    return jnp.where(x >= 0, x, 0.2 * x)


def _rows(shape):
    nd = len(shape)
    return pl.BlockSpec(shape, lambda i: (i,) + (0,) * (nd - 1))


def _full(shape):
    nd = len(shape)
    return pl.BlockSpec(shape, lambda i: (0,) * nd)


def _dot(a, b):
    return jnp.dot(a, b, precision=PREC)


def _enc_body(x, w1, b1, w2, b2, o):
    o[...] = _dot(jax.nn.relu(_dot(x[...], w1[...]) + b1[...]), w2[...]) + b2[...]


def _enc_edges_body(x, w1, b1, w2, b2, we, ae, oe, ob):
    e = _dot(jax.nn.relu(_dot(x[...], w1[...]) + b1[...]), w2[...]) + b2[...]
    oe[...] = e
    ob[...] = jnp.sum(_lr(_dot(e, we[...])) * ae[...], -1, keepdims=True)


def _node_pre_body(h, wq, wk, wm, aqr, akr, oaq, oak, om):
    hh = h[...]
    oaq[...] = jnp.sum(_lr(_dot(hh, wq[...])) * aqr[...], -1, keepdims=True)
    oak[...] = jnp.sum(_lr(_dot(hh, wk[...])) * akr[...], -1, keepdims=True)
    om[...] = _dot(hh, wm[...])


def _node_post_body(h, agg, sr, aqc, wself, bself, g, b, wa, wb, ohn, oa, ob):
    hh = h[...]
    a = agg[...] / (sr[...] + 1e-12 * jnp.exp(-aqc[...]))
    hn = jax.nn.relu(_dot(hh, wself[...]) + bself[...] + a)
    x = hn + hh
    mu = jnp.mean(x, -1, keepdims=True)
    xc = x - mu
    var = jnp.mean(xc * xc, -1, keepdims=True)
    hn2 = xc / jnp.sqrt(var + 1e-5) * g[...] + b[...]
    ohn[...] = hn2
    oa[...] = _dot(hn2, wa[...])
    ob[...] = _dot(hn2, wb[...])


def _edge_post_beta_body(e, g, wee, beu, wen, aen, oe, ob):
    gg = g[...]
    en = jax.nn.relu(gg[:, 0:H] + gg[:, H:2 * H]
                     + _dot(e[...], wee[...]) + beu[...]) + e[...]
    oe[...] = en
    ob[...] = jnp.sum(_lr(_dot(en, wen[...])) * aen[...], -1, keepdims=True)


def _edge_post_body(e, g, wee, beu, oe):
    gg = g[...]
    oe[...] = jax.nn.relu(gg[:, 0:H] + gg[:, H:2 * H]
                          + _dot(e[...], wee[...]) + beu[...]) + e[...]


def _node_final_body(h, ws, wd, ou, ov):
    hh = h[...]
    ou[...] = _dot(hh, ws[...])
    ov[...] = _dot(hh, wd[...])


def _final_body(e, g, w1e, b1, w2r, b2, os):
    gg = g[...]
    hid = jax.nn.relu(gg[:, 0:H] + gg[:, H:2 * H]
                      + _dot(e[...], w1e[...]) + b1[...])
    os[...] = jax.nn.sigmoid(jnp.sum(hid * w2r[...], -1, keepdims=True) + b2[...])


def _tc(body, grid, in_specs, out_specs, out_shape):
    return pl.pallas_call(
        body, grid=(grid,), in_specs=in_specs, out_specs=out_specs,
        out_shape=out_shape)


_MESH = plsc.VectorSubcoreMesh(core_axis_name="c", subcore_axis_name="s")
_SC_PARAMS = pltpu.CompilerParams(
    needs_layout_passes=False, use_tc_tiling_on_sc=False)


def _sc_pair_gather_body(ta, tb, sdr, g, sd0, sd1, rab0, rab1,
                         ds0, ds1, gs0, gs1, ws0, ws1):
    c = lax.axis_index("c")
    s = lax.axis_index("s")
    wid = s * 2 + c
    wbase = wid * 200

    def fire_sd(row0, sd, dsem):
        pltpu.async_copy(sdr.at[pl.ds(row0, 4)], sd, dsem)

    def wait_sd(sd, dsem):
        pltpu.make_async_copy(sdr.at[pl.ds(0, 4)], sd, dsem).wait()

    def fire_g(half, sd, rab, gs):
        for j in range(2):
            pltpu.async_copy(ta.at[sd.at[half * 2 + j, 0]],
                             rab.at[pl.ds(j * 128, 128)], gs)
            pltpu.async_copy(tb.at[sd.at[half * 2 + j, 1]],
                             rab.at[pl.ds(256 + j * 128, 128)], gs)

    def wait_g(rab, gs):
        pltpu.make_async_copy(ta.at[pl.ds(0, 512)], rab, gs).wait()

    def fire_w(eb, rab, ws):
        pltpu.async_copy(rab.at[pl.ds(0, 256)],
                         g.at[pl.ds(eb, 256), pl.ds(0, H)], ws)
        pltpu.async_copy(rab.at[pl.ds(256, 256)],
                         g.at[pl.ds(eb, 256), pl.ds(H, H)], ws)

    def wait_w(rab, ws):
        pltpu.make_async_copy(rab, g.at[pl.ds(0, 512), pl.ds(0, H)], ws).wait()

    pltpu.sync_copy(sdr.at[pl.ds(wbase, 4)], sd0)
    fire_g(0, sd0, rab0, gs0)
    fire_g(1, sd0, rab1, gs1)

    def pair(p, carry):
        row0 = wbase + p * 4
        eb0 = pl.multiple_of(row0 * 128, 256)
        eb1 = pl.multiple_of((row0 + 2) * 128, 256)
        wait_g(rab0, gs0)
        fire_w(eb0, rab0, ws0)
        wait_g(rab1, gs1)
        fire_w(eb1, rab1, ws1)

        @pl.when(p < 49)
        def _pref():
            pltpu.sync_copy(sdr.at[pl.ds(row0 + 4, 4)], sd0)
            wait_w(rab0, ws0)
            fire_g(0, sd0, rab0, gs0)
            wait_w(rab1, ws1)
            fire_g(1, sd0, rab1, gs1)

        return carry

    lax.fori_loop(0, 50, pair, 0)
    wait_w(rab0, ws0)
    wait_w(rab1, ws1)


_sc_pair_gather = pl.kernel(
    _sc_pair_gather_body,
    out_type=jax.ShapeDtypeStruct((EPAD, 2 * H), _f32),
    mesh=_MESH,
    compiler_params=_SC_PARAMS,
    scratch_types=[
        pltpu.VMEM((4, 2, 128), _i32),
        pltpu.VMEM((4, 2, 128), _i32),
        pltpu.VMEM((512, H), _f32),
        pltpu.VMEM((512, H), _f32),
        pltpu.SemaphoreType.DMA,
        pltpu.SemaphoreType.DMA,
        pltpu.SemaphoreType.DMA,
        pltpu.SemaphoreType.DMA,
        pltpu.SemaphoreType.DMA,
        pltpu.SemaphoreType.DMA,
    ],
)


def _make_sc_attn_body():
    def body(sdr, akt, hmsg, s_out, agg_out,
             sd, akv0, akv1, sv0, sv1, sf0, sf1, li0, li1, rows0, rows1, zs,
             s_sh, agg_sh, gs0, gs1, ss0, ss1):
        c = lax.axis_index("c")
        s = lax.axis_index("s")
        base = c * HALF
        sbase = s * 400
        zero16 = jnp.zeros((16,), _f32)
        iota16 = lax.iota(_i32, 16)

        def zrow(i, carry):
            for jj in range(4):
                rows0[i, pl.ds(jj * 16, 16)] = zero16
            return carry

        lax.fori_loop(0, 128, zrow, 0)

        def zz(i, carry):
            zs[pl.ds(i * 16, 16)] = zero16
            return carry

        lax.fori_loop(0, 98, zz, 0)

        r0 = pl.multiple_of(s * 1568, 8)
        for t in range(12):
            pltpu.sync_copy(rows0, agg_sh.at[pl.ds(r0 + t * 128, 128)])
        pltpu.sync_copy(rows0.at[pl.ds(0, 32)], agg_sh.at[pl.ds(r0 + 1536, 32)])
        pltpu.sync_copy(zs, s_sh.at[pl.ds(r0, 1568)])
        plsc.subcore_barrier()

        def fire_g(half, akv, rows, gs):
            pltpu.async_copy(akt.at[sd.at[half, 0]], akv.at[0], gs)
            pltpu.async_copy(hmsg.at[sd.at[half, 0]], rows, gs)

        def wait_g(akv, rows, gs):
            pltpu.make_async_copy(akt.at[pl.ds(0, 128)], akv.at[0], gs).wait()
            pltpu.make_async_copy(hmsg.at[pl.ds(0, 128)], rows, gs).wait()

        def fire_s(akv, rows, sv, li, ss):
            pltpu.async_copy(rows, agg_sh.at[li.at[0]], ss, add=True)
            pltpu.async_copy(sv.at[0], s_sh.at[li.at[0]], ss, add=True)

        def wait_s(rows, sv, li, ss):
            pltpu.make_async_copy(rows, agg_sh.at[li.at[0]], ss).wait()
            pltpu.make_async_copy(sv.at[0], s_sh.at[li.at[0]], ss).wait()

        def compute(half, akv, rows, sv, sf, li):
            def grp(i, carry):
                dsl = pl.ds(i * 16, 16)
                bv = plsc.bitcast(sd[half, 2, dsl], _f32)
                sval = jnp.exp(akv[0, dsl] + bv)
                sv[0, dsl] = sval
                sf[pl.ds(i * 16, 16)] = sval
                d = sd[half, 1, dsl] - base
                owned = (d >= 0) & (d < HALF)
                tr = HALF + iota16 + (i % 4) * 16
                li[0, dsl] = jnp.where(owned, d, tr)
                return carry

            lax.fori_loop(0, 8, grp, 0)

            def scale(eq, carry):
                for d in range(4):
                    ei = eq * 4 + d
                    spl = plsc.load_gather(sf, [jnp.full((16,), ei, _i32)])
                    for jj in range(4):
                        dsl = pl.ds(jj * 16, 16)
                        rows[ei, dsl] = rows[ei, dsl] * spl
                return carry

            lax.fori_loop(0, 32, scale, 0)

        pltpu.sync_copy(sdr.at[pl.ds(sbase, 2)], sd)
        fire_g(0, akv0, rows0, gs0)
        fire_g(1, akv1, rows1, gs1)

        def pair(p, carry):
            row0 = sbase + p * 2
            wait_g(akv0, rows0, gs0)
            compute(0, akv0, rows0, sv0, sf0, li0)
            fire_s(akv0, rows0, sv0, li0, ss0)
            wait_g(akv1, rows1, gs1)
            compute(1, akv1, rows1, sv1, sf1, li1)
            fire_s(akv1, rows1, sv1, li1, ss1)

            @pl.when(p < 199)
            def _pref():
                pltpu.sync_copy(sdr.at[pl.ds(row0 + 2, 2)], sd)
                wait_s(rows0, sv0, li0, ss0)
                fire_g(0, akv0, rows0, gs0)
                wait_s(rows1, sv1, li1, ss1)
                fire_g(1, akv1, rows1, gs1)

            return carry

        lax.fori_loop(0, 200, pair, 0)
        wait_s(rows0, sv0, li0, ss0)
        wait_s(rows1, sv1, li1, ss1)
        plsc.subcore_barrier()

        o0 = pl.multiple_of(s * 1560, 8)
        pltpu.sync_copy(agg_sh.at[pl.ds(o0, 1560)],
                        agg_out.at[pl.ds(base + o0, 1560)])
        pltpu.sync_copy(s_sh.at[pl.ds(o0, 1560)],
                        s_out.at[pl.ds(base + o0, 1560)])

        @pl.when(s == 0)
        def _tail():
            pltpu.sync_copy(agg_sh.at[pl.ds(24960, 40)],
                            agg_out.at[pl.ds(base + 24960, 40)])
            pltpu.sync_copy(s_sh.at[pl.ds(24960, 40)],
                            s_out.at[pl.ds(base + 24960, 40)])

    return body


_sc_attn_agg = pl.kernel(
    _make_sc_attn_body(),
    out_type=[jax.ShapeDtypeStruct((N,), _f32),
              jax.ShapeDtypeStruct((N, H), _f32)],
    mesh=_MESH,
    compiler_params=_SC_PARAMS,
    scratch_types=[
        pltpu.VMEM((2, 3, 128), _i32),
        pltpu.VMEM((1, 128), _f32),
        pltpu.VMEM((1, 128), _f32),
        pltpu.VMEM((1, 128), _f32),
        pltpu.VMEM((1, 128), _f32),
        pltpu.VMEM((128,), _f32),
        pltpu.VMEM((128,), _f32),
        pltpu.VMEM((1, 128), _i32),
        pltpu.VMEM((1, 128), _i32),
        pltpu.VMEM((128, H), _f32),
        pltpu.VMEM((128, H), _f32),
        pltpu.VMEM((1568,), _f32),
        pltpu.VMEM_SHARED((TBL,), _f32),
        pltpu.VMEM_SHARED((TBL, H), _f32),
        pltpu.SemaphoreType.DMA,
        pltpu.SemaphoreType.DMA,
        pltpu.SemaphoreType.DMA,
        pltpu.SemaphoreType.DMA,
    ],
)


def kernel(node_features, edge_features, edge_index, params):
    p = params
    src = edge_index[0].astype(_i32)
    dst = edge_index[1].astype(_i32)
    srcr = jnp.pad(src, (0, EPAD - E)).reshape(ER, 128)
    dstr = jnp.pad(dst, (0, EPAD - E), constant_values=N).reshape(ER, 128)
    sdr2 = jnp.stack([srcr, dstr], axis=1)

    h = _tc(_enc_body, N // BN,
            [_rows((BN, 8)), _full((8, H)), _full((1, H)), _full((H, H)),
             _full((1, H))],
            _rows((BN, H)), jax.ShapeDtypeStruct((N, H), _f32))(
        node_features, p['ne_w1'], p['ne_b1'].reshape(1, H),
        p['ne_w2'], p['ne_b2'].reshape(1, H))

    lp0 = p['layers'][0]
    e, beta = _tc(_enc_edges_body, E // BE,
                  [_rows((BE, 8)), _full((8, H)), _full((1, H)), _full((H, H)),
                   _full((1, H)), _full((H, H)), _full((1, H))],
                  [_rows((BE, H)), _rows((BE, 1))],
                  [jax.ShapeDtypeStruct((E, H), _f32),
                   jax.ShapeDtypeStruct((E, 1), _f32)])(
        edge_features, p['ee_w1'], p['ee_b1'].reshape(1, H),
        p['ee_w2'], p['ee_b2'].reshape(1, H),
        lp0['We'], lp0['a'][128:192, 0].reshape(1, H))

    for li in range(3):
        lp = p['layers'][li]
        aq, ak, hmsg = _tc(
            _node_pre_body, N // BN,
            [_rows((BN, H)), _full((H, H)), _full((H, H)), _full((H, H)),
             _full((1, H)), _full((1, H))],
            [_rows((BN, 1)), _rows((BN, 1)), _rows((BN, H))],
            [jax.ShapeDtypeStruct((N, 1), _f32),
             jax.ShapeDtypeStruct((N, 1), _f32),
             jax.ShapeDtypeStruct((N, H), _f32)])(
            h, lp['Wq'], lp['Wk'], lp['Wmsg'],
            lp['a'][0:64, 0].reshape(1, H), lp['a'][64:128, 0].reshape(1, H))

        betar = lax.bitcast_convert_type(
            jnp.pad(beta[:, 0], (0, EPAD - E)).reshape(ER, 128), _i32)
        sdr3 = jnp.stack([srcr, dstr, betar], axis=1)
        s_sum, agg_raw = _sc_attn_agg(sdr3, ak[:, 0], hmsg)

        hn, ta, tb = _tc(
            _node_post_body, N // BN,
            [_rows((BN, H)), _rows((BN, H)), _rows((BN, 1)), _rows((BN, 1)),
             _full((H, H)), _full((1, H)), _full((1, H)), _full((1, H)),
             _full((H, H)), _full((H, H))],
            [_rows((BN, H)), _rows((BN, H)), _rows((BN, H))],
            [jax.ShapeDtypeStruct((N, H), _f32),
             jax.ShapeDtypeStruct((N, H), _f32),
             jax.ShapeDtypeStruct((N, H), _f32)])(
            h, agg_raw, s_sum.reshape(N, 1), aq, lp['Wself'],
            lp['bself'].reshape(1, H), lp['ln_g'].reshape(1, H),
            lp['ln_b'].reshape(1, H), lp['Weu'][0:64], lp['Weu'][64:128])

        g = _sc_pair_gather(
            ta, jnp.pad(tb, ((0, NPAD - N), (0, 0))), sdr2)

        if li < 2:
            lpn = p['layers'][li + 1]
            e, beta = _tc(
                _edge_post_beta_body, E // BE,
                [_rows((BE, H)), _rows((BE, 2 * H)), _full((H, H)),
                 _full((1, H)), _full((H, H)), _full((1, H))],
                [_rows((BE, H)), _rows((BE, 1))],
                [jax.ShapeDtypeStruct((E, H), _f32),
                 jax.ShapeDtypeStruct((E, 1), _f32)])(
                e, g, lp['Weu'][128:192], lp['beu'].reshape(1, H),
                lpn['We'], lpn['a'][128:192, 0].reshape(1, H))
        else:
            e = _tc(
                _edge_post_body, E // BE,
                [_rows((BE, H)), _rows((BE, 2 * H)), _full((H, H)),
                 _full((1, H))],
                _rows((BE, H)), jax.ShapeDtypeStruct((E, H), _f32))(
                e, g, lp['Weu'][128:192], lp['beu'].reshape(1, H))
        h = hn

    u, v = _tc(_node_final_body, N // BN,
               [_rows((BN, H)), _full((H, H)), _full((H, H))],
               [_rows((BN, H)), _rows((BN, H))],
               [jax.ShapeDtypeStruct((N, H), _f32),
                jax.ShapeDtypeStruct((N, H), _f32)])(
        h, p['em_w1'][0:64], p['em_w1'][64:128])

    gf = _sc_pair_gather(u, jnp.pad(v, ((0, NPAD - N), (0, 0))), sdr2)

    scores = _tc(
        _final_body, E // BE,
        [_rows((BE, H)), _rows((BE, 2 * H)), _full((H, H)),
         _full((1, H)), _full((1, H)), _full((1, 1))],
        _rows((BE, 1)), jax.ShapeDtypeStruct((E, 1), _f32))(
        e, gf, p['em_w1'][128:192], p['em_b1'].reshape(1, H),
        p['em_w2'][:, 0].reshape(1, H), p['em_b2'].reshape(1, 1))

    return scores[:, 0]


# speedup vs baseline: 5.1146x; 1.0033x over previous
import functools

import jax
import jax.numpy as jnp
from jax import lax
from jax.experimental import pallas as pl
from jax.experimental.pallas import tpu as pltpu
from jax.experimental.pallas import tpu_sc as plsc

N = 50000
E = 800000
H = 64
NPAD = 50048
EPAD = 819200
ER = EPAD // 128
BN = 2000
BE = 8000
HALF = 25000
TBL = 25088
PREC = lax.Precision.DEFAULT

_f32 = jnp.float32
_i32 = jnp.int32


def _lr(x):
    return jnp.where(x >= 0, x, 0.2 * x)


def _rows(shape):
    nd = len(shape)
    return pl.BlockSpec(shape, lambda i: (i,) + (0,) * (nd - 1))


def _full(shape):
    nd = len(shape)
    return pl.BlockSpec(shape, lambda i: (0,) * nd)


def _dot(a, b):
    return jnp.dot(a, b, precision=PREC)


def _enc_body(x, w1, b1, w2, b2, o):
    o[...] = _dot(jax.nn.relu(_dot(x[...], w1[...]) + b1[...]), w2[...]) + b2[...]


def _enc_edges_body(x, w1, b1, w2, b2, we, ae, oe, ob):
    e = _dot(jax.nn.relu(_dot(x[...], w1[...]) + b1[...]), w2[...]) + b2[...]
    oe[...] = e
    ob[...] = jnp.sum(_lr(_dot(e, we[...])) * ae[...], -1, keepdims=True)


def _node_pre_body(h, wq, wk, wm, aqr, akr, oaq, oak, om):
    hh = h[...]
    oaq[...] = jnp.sum(_lr(_dot(hh, wq[...])) * aqr[...], -1, keepdims=True)
    oak[...] = jnp.sum(_lr(_dot(hh, wk[...])) * akr[...], -1, keepdims=True)
    om[...] = _dot(hh, wm[...])


def _node_post(h, agg, sr, aqc, wself, bself, g, b):
    hh = h[...]
    a = agg[...] / (sr[...] + 1e-12 * jnp.exp(-aqc[...]))
    hn = jax.nn.relu(_dot(hh, wself[...]) + bself[...] + a)
    x = hn + hh
    mu = jnp.mean(x, -1, keepdims=True)
    xc = x - mu
    var = jnp.mean(xc * xc, -1, keepdims=True)
    return xc / jnp.sqrt(var + 1e-5) * g[...] + b[...]


def _node_post_pre_body(h, agg, sr, aqc, wself, bself, g, b, wa, wb,
                        wqn, wkn, wmn, aqrn, akrn,
                        ohn, oa, ob, oaq, oak, om):
    hn2 = _node_post(h, agg, sr, aqc, wself, bself, g, b)
    ohn[...] = hn2
    oa[...] = _dot(hn2, wa[...])
    ob[...] = _dot(hn2, wb[...])
    oaq[...] = jnp.sum(_lr(_dot(hn2, wqn[...])) * aqrn[...], -1, keepdims=True)
    oak[...] = jnp.sum(_lr(_dot(hn2, wkn[...])) * akrn[...], -1, keepdims=True)
    om[...] = _dot(hn2, wmn[...])


def _node_post_final_body(h, agg, sr, aqc, wself, bself, g, b, wa, wb,
                          ws, wd, ohn, oa, ob, ou, ov):
    hn2 = _node_post(h, agg, sr, aqc, wself, bself, g, b)
    ohn[...] = hn2
    oa[...] = _dot(hn2, wa[...])
    ob[...] = _dot(hn2, wb[...])
    ou[...] = _dot(hn2, ws[...])
    ov[...] = _dot(hn2, wd[...])


def _edge_post_beta_body(e, g, wee, beu, wen, aen, oe, ob):
    gg = g[...]
    en = jax.nn.relu(gg[:, 0:H] + gg[:, H:2 * H]
                     + _dot(e[...], wee[...]) + beu[...]) + e[...]
    oe[...] = en
    ob[...] = jnp.sum(_lr(_dot(en, wen[...])) * aen[...], -1, keepdims=True)


def _edge_post_body(e, g, wee, beu, oe):
    gg = g[...]
    oe[...] = jax.nn.relu(gg[:, 0:H] + gg[:, H:2 * H]
                          + _dot(e[...], wee[...]) + beu[...]) + e[...]


def _node_final_body(h, ws, wd, ou, ov):
    hh = h[...]
    ou[...] = _dot(hh, ws[...])
    ov[...] = _dot(hh, wd[...])


def _final_body(e, g, w1e, b1, w2r, b2, os):
    gg = g[...]
    hid = jax.nn.relu(gg[:, 0:H] + gg[:, H:2 * H]
                      + _dot(e[...], w1e[...]) + b1[...])
    os[...] = jax.nn.sigmoid(jnp.sum(hid * w2r[...], -1, keepdims=True) + b2[...])


def _tc(body, grid, in_specs, out_specs, out_shape):
    return pl.pallas_call(
        body, grid=(grid,), in_specs=in_specs, out_specs=out_specs,
        out_shape=out_shape)


_MESH = plsc.VectorSubcoreMesh(core_axis_name="c", subcore_axis_name="s")
_SC_PARAMS = pltpu.CompilerParams(
    needs_layout_passes=False, use_tc_tiling_on_sc=False)


def _sc_pair_gather_body(ta, tb, sdr, g, sd0, sd1, rab0, rab1,
                         ds0, ds1, gs0, gs1, ws0, ws1):
    c = lax.axis_index("c")
    s = lax.axis_index("s")
    wid = s * 2 + c
    wbase = wid * 200

    def fire_sd(row0, sd, dsem):
        pltpu.async_copy(sdr.at[pl.ds(row0, 4)], sd, dsem)

    def wait_sd(sd, dsem):
        pltpu.make_async_copy(sdr.at[pl.ds(0, 4)], sd, dsem).wait()

    def fire_g(half, sd, rab, gs):
        for j in range(2):
            pltpu.async_copy(ta.at[sd.at[half * 2 + j, 0]],
                             rab.at[pl.ds(j * 128, 128)], gs)
            pltpu.async_copy(tb.at[sd.at[half * 2 + j, 1]],
                             rab.at[pl.ds(256 + j * 128, 128)], gs)

    def wait_g(rab, gs):
        pltpu.make_async_copy(ta.at[pl.ds(0, 512)], rab, gs).wait()

    def fire_w(eb, rab, ws):
        pltpu.async_copy(rab.at[pl.ds(0, 256)],
                         g.at[pl.ds(eb, 256), pl.ds(0, H)], ws)
        pltpu.async_copy(rab.at[pl.ds(256, 256)],
                         g.at[pl.ds(eb, 256), pl.ds(H, H)], ws)

    def wait_w(rab, ws):
        pltpu.make_async_copy(rab, g.at[pl.ds(0, 512), pl.ds(0, H)], ws).wait()

    pltpu.sync_copy(sdr.at[pl.ds(wbase, 4)], sd0)
    fire_g(0, sd0, rab0, gs0)
    fire_g(1, sd0, rab1, gs1)

    def pair(p, carry):
        row0 = wbase + p * 4
        eb0 = pl.multiple_of(row0 * 128, 256)
        eb1 = pl.multiple_of((row0 + 2) * 128, 256)
        wait_g(rab0, gs0)
        fire_w(eb0, rab0, ws0)
        wait_g(rab1, gs1)
        fire_w(eb1, rab1, ws1)

        @pl.when(p < 49)
        def _pref():
            pltpu.sync_copy(sdr.at[pl.ds(row0 + 4, 4)], sd0)
            wait_w(rab0, ws0)
            fire_g(0, sd0, rab0, gs0)
            wait_w(rab1, ws1)
            fire_g(1, sd0, rab1, gs1)

        return carry

    lax.fori_loop(0, 50, pair, 0)
    wait_w(rab0, ws0)
    wait_w(rab1, ws1)


_sc_pair_gather = pl.kernel(
    _sc_pair_gather_body,
    out_type=jax.ShapeDtypeStruct((EPAD, 2 * H), _f32),
    mesh=_MESH,
    compiler_params=_SC_PARAMS,
    scratch_types=[
        pltpu.VMEM((4, 2, 128), _i32),
        pltpu.VMEM((4, 2, 128), _i32),
        pltpu.VMEM((512, H), _f32),
        pltpu.VMEM((512, H), _f32),
        pltpu.SemaphoreType.DMA,
        pltpu.SemaphoreType.DMA,
        pltpu.SemaphoreType.DMA,
        pltpu.SemaphoreType.DMA,
        pltpu.SemaphoreType.DMA,
        pltpu.SemaphoreType.DMA,
    ],
)


def _make_sc_attn_body():
    def body(sdr, akt, hmsg, s_out, agg_out,
             sd, akv0, akv1, sv0, sv1, sf0, sf1, li0, li1, rows0, rows1, zs,
             s_sh, agg_sh, gs0, gs1, ss0, ss1):
        c = lax.axis_index("c")
        s = lax.axis_index("s")
        base = c * HALF
        sbase = s * 400
        zero16 = jnp.zeros((16,), _f32)
        iota16 = lax.iota(_i32, 16)

        def zrow(i, carry):
            for jj in range(4):
                rows0[i, pl.ds(jj * 16, 16)] = zero16
            return carry

        lax.fori_loop(0, 128, zrow, 0)

        def zz(i, carry):
            zs[pl.ds(i * 16, 16)] = zero16
            return carry

        lax.fori_loop(0, 98, zz, 0)

        r0 = pl.multiple_of(s * 1568, 8)
        for t in range(12):
            pltpu.sync_copy(rows0, agg_sh.at[pl.ds(r0 + t * 128, 128)])
        pltpu.sync_copy(rows0.at[pl.ds(0, 32)], agg_sh.at[pl.ds(r0 + 1536, 32)])
        pltpu.sync_copy(zs, s_sh.at[pl.ds(r0, 1568)])
        plsc.subcore_barrier()

        def fire_g(half, akv, rows, gs):
            pltpu.async_copy(akt.at[sd.at[half, 0]], akv.at[0], gs)
            pltpu.async_copy(hmsg.at[sd.at[half, 0]], rows, gs)

        def wait_g(akv, rows, gs):
            pltpu.make_async_copy(akt.at[pl.ds(0, 128)], akv.at[0], gs).wait()
            pltpu.make_async_copy(hmsg.at[pl.ds(0, 128)], rows, gs).wait()

        def fire_s(akv, rows, sv, li, ss):
            pltpu.async_copy(rows, agg_sh.at[li.at[0]], ss, add=True)
            pltpu.async_copy(sv.at[0], s_sh.at[li.at[0]], ss, add=True)

        def wait_s(rows, sv, li, ss):
            pltpu.make_async_copy(rows, agg_sh.at[li.at[0]], ss).wait()
            pltpu.make_async_copy(sv.at[0], s_sh.at[li.at[0]], ss).wait()

        def compute(half, akv, rows, sv, sf, li):
            def grp(i, carry):
                dsl = pl.ds(i * 16, 16)
                bv = plsc.bitcast(sd[half, 2, dsl], _f32)
                sval = jnp.exp(akv[0, dsl] + bv)
                sv[0, dsl] = sval
                sf[pl.ds(i * 16, 16)] = sval
                d = sd[half, 1, dsl] - base
                owned = (d >= 0) & (d < HALF)
                tr = HALF + iota16 + (i % 4) * 16
                li[0, dsl] = jnp.where(owned, d, tr)
                return carry

            lax.fori_loop(0, 8, grp, 0)

            def scale(eq, carry):
                for d in range(4):
                    ei = eq * 4 + d
                    spl = plsc.load_gather(sf, [jnp.full((16,), ei, _i32)])
                    for jj in range(4):
                        dsl = pl.ds(jj * 16, 16)
                        rows[ei, dsl] = rows[ei, dsl] * spl
                return carry

            lax.fori_loop(0, 32, scale, 0)

        pltpu.sync_copy(sdr.at[pl.ds(sbase, 2)], sd)
        fire_g(0, akv0, rows0, gs0)
        fire_g(1, akv1, rows1, gs1)

        def pair(p, carry):
            row0 = sbase + p * 2
            wait_g(akv0, rows0, gs0)
            compute(0, akv0, rows0, sv0, sf0, li0)
            fire_s(akv0, rows0, sv0, li0, ss0)
            wait_g(akv1, rows1, gs1)
            compute(1, akv1, rows1, sv1, sf1, li1)
            fire_s(akv1, rows1, sv1, li1, ss1)

            @pl.when(p < 199)
            def _pref():
                pltpu.sync_copy(sdr.at[pl.ds(row0 + 2, 2)], sd)
                wait_s(rows0, sv0, li0, ss0)
                fire_g(0, akv0, rows0, gs0)
                wait_s(rows1, sv1, li1, ss1)
                fire_g(1, akv1, rows1, gs1)

            return carry

        lax.fori_loop(0, 200, pair, 0)
        wait_s(rows0, sv0, li0, ss0)
        wait_s(rows1, sv1, li1, ss1)
        plsc.subcore_barrier()

        o0 = pl.multiple_of(s * 1560, 8)
        pltpu.sync_copy(agg_sh.at[pl.ds(o0, 1560)],
                        agg_out.at[pl.ds(base + o0, 1560)])
        pltpu.sync_copy(s_sh.at[pl.ds(o0, 1560)],
                        s_out.at[pl.ds(base + o0, 1560)])

        @pl.when(s == 0)
        def _tail():
            pltpu.sync_copy(agg_sh.at[pl.ds(24960, 40)],
                            agg_out.at[pl.ds(base + 24960, 40)])
            pltpu.sync_copy(s_sh.at[pl.ds(24960, 40)],
                            s_out.at[pl.ds(base + 24960, 40)])

    return body


_sc_attn_agg = pl.kernel(
    _make_sc_attn_body(),
    out_type=[jax.ShapeDtypeStruct((N,), _f32),
              jax.ShapeDtypeStruct((N, H), _f32)],
    mesh=_MESH,
    compiler_params=_SC_PARAMS,
    scratch_types=[
        pltpu.VMEM((2, 3, 128), _i32),
        pltpu.VMEM((1, 128), _f32),
        pltpu.VMEM((1, 128), _f32),
        pltpu.VMEM((1, 128), _f32),
        pltpu.VMEM((1, 128), _f32),
        pltpu.VMEM((128,), _f32),
        pltpu.VMEM((128,), _f32),
        pltpu.VMEM((1, 128), _i32),
        pltpu.VMEM((1, 128), _i32),
        pltpu.VMEM((128, H), _f32),
        pltpu.VMEM((128, H), _f32),
        pltpu.VMEM((1568,), _f32),
        pltpu.VMEM_SHARED((TBL,), _f32),
        pltpu.VMEM_SHARED((TBL, H), _f32),
        pltpu.SemaphoreType.DMA,
        pltpu.SemaphoreType.DMA,
        pltpu.SemaphoreType.DMA,
        pltpu.SemaphoreType.DMA,
    ],
)


def kernel(node_features, edge_features, edge_index, params):
    p = params
    src = edge_index[0].astype(_i32)
    dst = edge_index[1].astype(_i32)
    srcr = jnp.pad(src, (0, EPAD - E)).reshape(ER, 128)
    dstr = jnp.pad(dst, (0, EPAD - E), constant_values=N).reshape(ER, 128)
    sdr2 = jnp.stack([srcr, dstr], axis=1)

    h = _tc(_enc_body, N // BN,
            [_rows((BN, 8)), _full((8, H)), _full((1, H)), _full((H, H)),
             _full((1, H))],
            _rows((BN, H)), jax.ShapeDtypeStruct((N, H), _f32))(
        node_features, p['ne_w1'], p['ne_b1'].reshape(1, H),
        p['ne_w2'], p['ne_b2'].reshape(1, H))

    lp0 = p['layers'][0]
    e, beta = _tc(_enc_edges_body, E // BE,
                  [_rows((BE, 8)), _full((8, H)), _full((1, H)), _full((H, H)),
                   _full((1, H)), _full((H, H)), _full((1, H))],
                  [_rows((BE, H)), _rows((BE, 1))],
                  [jax.ShapeDtypeStruct((E, H), _f32),
                   jax.ShapeDtypeStruct((E, 1), _f32)])(
        edge_features, p['ee_w1'], p['ee_b1'].reshape(1, H),
        p['ee_w2'], p['ee_b2'].reshape(1, H),
        lp0['We'], lp0['a'][128:192, 0].reshape(1, H))

    lp0 = p['layers'][0]
    aq, ak, hmsg = _tc(
        _node_pre_body, N // BN,
        [_rows((BN, H)), _full((H, H)), _full((H, H)), _full((H, H)),
         _full((1, H)), _full((1, H))],
        [_rows((BN, 1)), _rows((BN, 1)), _rows((BN, H))],
        [jax.ShapeDtypeStruct((N, 1), _f32),
         jax.ShapeDtypeStruct((N, 1), _f32),
         jax.ShapeDtypeStruct((N, H), _f32)])(
        h, lp0['Wq'], lp0['Wk'], lp0['Wmsg'],
        lp0['a'][0:64, 0].reshape(1, H), lp0['a'][64:128, 0].reshape(1, H))

    nh = jax.ShapeDtypeStruct((N, H), _f32)
    n1 = jax.ShapeDtypeStruct((N, 1), _f32)
    for li in range(3):
        lp = p['layers'][li]
        betar = lax.bitcast_convert_type(
            jnp.pad(beta[:, 0], (0, EPAD - E)).reshape(ER, 128), _i32)
        sdr3 = jnp.stack([srcr, dstr, betar], axis=1)
        s_sum, agg_raw = _sc_attn_agg(sdr3, ak[:, 0], hmsg)

        post_args = (h, agg_raw, s_sum.reshape(N, 1), aq, lp['Wself'],
                     lp['bself'].reshape(1, H), lp['ln_g'].reshape(1, H),
                     lp['ln_b'].reshape(1, H), lp['Weu'][0:64],
                     lp['Weu'][64:128])
        post_specs = [_rows((BN, H)), _rows((BN, H)), _rows((BN, 1)),
                      _rows((BN, 1)), _full((H, H)), _full((1, H)),
                      _full((1, H)), _full((1, H)), _full((H, H)),
                      _full((H, H))]
        if li < 2:
            lpn = p['layers'][li + 1]
            hn, ta, tb, aq, ak, hmsg = _tc(
                _node_post_pre_body, N // BN,
                post_specs + [_full((H, H)), _full((H, H)), _full((H, H)),
                              _full((1, H)), _full((1, H))],
                [_rows((BN, H)), _rows((BN, H)), _rows((BN, H)),
                 _rows((BN, 1)), _rows((BN, 1)), _rows((BN, H))],
                [nh, nh, nh, n1, n1, nh])(
                *post_args, lpn['Wq'], lpn['Wk'], lpn['Wmsg'],
                lpn['a'][0:64, 0].reshape(1, H),
                lpn['a'][64:128, 0].reshape(1, H))
        else:
            hn, ta, tb, u, v = _tc(
                _node_post_final_body, N // BN,
                post_specs + [_full((H, H)), _full((H, H))],
                [_rows((BN, H)), _rows((BN, H)), _rows((BN, H)),
                 _rows((BN, H)), _rows((BN, H))],
                [nh, nh, nh, nh, nh])(
                *post_args, p['em_w1'][0:64], p['em_w1'][64:128])

        g = _sc_pair_gather(
            ta, jnp.pad(tb, ((0, NPAD - N), (0, 0))), sdr2)

        if li < 2:
            e, beta = _tc(
                _edge_post_beta_body, E // BE,
                [_rows((BE, H)), _rows((BE, 2 * H)), _full((H, H)),
                 _full((1, H)), _full((H, H)), _full((1, H))],
                [_rows((BE, H)), _rows((BE, 1))],
                [jax.ShapeDtypeStruct((E, H), _f32),
                 jax.ShapeDtypeStruct((E, 1), _f32)])(
                e, g, lp['Weu'][128:192], lp['beu'].reshape(1, H),
                lpn['We'], lpn['a'][128:192, 0].reshape(1, H))
        else:
            e = _tc(
                _edge_post_body, E // BE,
                [_rows((BE, H)), _rows((BE, 2 * H)), _full((H, H)),
                 _full((1, H))],
                _rows((BE, H)), jax.ShapeDtypeStruct((E, H), _f32))(
                e, g, lp['Weu'][128:192], lp['beu'].reshape(1, H))
        h = hn

    gf = _sc_pair_gather(u, jnp.pad(v, ((0, NPAD - N), (0, 0))), sdr2)

    scores = _tc(
        _final_body, E // BE,
        [_rows((BE, H)), _rows((BE, 2 * H)), _full((H, H)),
         _full((1, H)), _full((1, H)), _full((1, 1))],
        _rows((BE, 1)), jax.ShapeDtypeStruct((E, 1), _f32))(
        e, gf, p['em_w1'][128:192], p['em_b1'].reshape(1, H),
        p['em_w2'][:, 0].reshape(1, H), p['em_b2'].reshape(1, 1))

    return scores[:, 0]
